# baseline (speedup 1.0000x reference)
import functools
import math

import jax
import jax.numpy as jnp
from jax import lax
from jax.experimental import pallas as pl
from jax.experimental.pallas import tpu as pltpu
from jax.experimental.pallas import tpu_sc as plsc

_EMB = 1024
_SEQ = 2048
_BATCH = 4
_ROWS = _SEQ * _BATCH
_NC, _NS = 2, 16
_NW = _NC * _NS
_RPW = _ROWS // _NW
_C = 32
_NCHUNK = _RPW // _C
_PC = _C // _BATCH
_LANES = 16
_SCALE = math.sqrt(_EMB)


def _sc_embed(tok_flat, table, pe):
    mesh = plsc.VectorSubcoreMesh(core_axis_name="c", subcore_axis_name="s")

    @functools.partial(
        pl.kernel,
        out_type=jax.ShapeDtypeStruct((_ROWS, _EMB), jnp.float32),
        mesh=mesh,
        scratch_types=[
            pltpu.VMEM((_RPW,), jnp.int32),
            pltpu.VMEM((2, _C, _EMB), jnp.float32),
            pltpu.VMEM((2, _PC, _EMB), jnp.float32),
            pltpu.SemaphoreType.DMA,
            pltpu.SemaphoreType.DMA,
            pltpu.SemaphoreType.DMA,
        ],
    )
    def k(tok_hbm, table_hbm, pe_hbm, out_hbm, idx_v, rows2, pos2, gsem, psem, osem):
        wid = lax.axis_index("s") * _NC + lax.axis_index("c")
        base = wid * _RPW
        pltpu.sync_copy(tok_hbm.at[pl.ds(pl.multiple_of(base, _RPW), _RPW)], idx_v)

        def issue(g, slot):
            ioff = pl.multiple_of(g * _C, _C)
            off = pl.multiple_of(base + g * _C, _C)
            pltpu.async_copy(
                table_hbm.at[idx_v.at[pl.ds(ioff, _C)]], rows2.at[slot], gsem)
            poff = pl.multiple_of(off // _BATCH, _PC)
            pltpu.async_copy(pe_hbm.at[pl.ds(poff, _PC)], pos2.at[slot], psem)

        issue(0, 0)

        def wait_out():
            pltpu.make_async_copy(
                rows2.at[0], out_hbm.at[pl.ds(0, _C)], osem).wait()

        def chunk(g, carry):
            b = lax.rem(g, 2)
            nxt = 1 - b

            @pl.when(g + 1 < _NCHUNK)
            def _prefetch():
                @pl.when(g >= 1)
                def _drain_prev_out():
                    wait_out()
                issue(g + 1, nxt)

            pltpu.make_async_copy(
                table_hbm.at[idx_v.at[pl.ds(0, _C)]], rows2.at[b], gsem).wait()
            pltpu.make_async_copy(
                pe_hbm.at[pl.ds(0, _PC)], pos2.at[b], psem).wait()

            def quad(q, c2):
                for j in range(_EMB // _LANES):
                    sl = pl.ds(j * _LANES, _LANES)
                    pv = pos2[b, q, sl]
                    for t in range(_BATCH):
                        r = q * _BATCH + t
                        rows2[b, r, sl] = rows2[b, r, sl] * _SCALE + pv
                return c2

            lax.fori_loop(0, _PC, quad, 0)
            off = pl.multiple_of(base + g * _C, _C)
            pltpu.async_copy(rows2.at[b], out_hbm.at[pl.ds(off, _C)], osem)
            return carry

        lax.fori_loop(0, _NCHUNK, chunk, 0)
        wait_out()
        wait_out()

    return k(tok_flat, table, pe)


def kernel(tokens, table, pos_embedding):
    tok_flat = tokens.reshape(-1).astype(jnp.int32)
    pe = pos_embedding.reshape(pos_embedding.shape[0], _EMB)
    out = _sc_embed(tok_flat, table, pe)
    return out.reshape(_SEQ, _BATCH, _EMB)


# speedup vs baseline: 1.9674x; 1.9674x over previous
import functools
import math

import jax
import jax.numpy as jnp
from jax import lax
from jax.experimental import pallas as pl
from jax.experimental.pallas import tpu as pltpu
from jax.experimental.pallas import tpu_sc as plsc

_EMB = 1024
_SEQ = 2048
_BATCH = 4
_ROWS = _SEQ * _BATCH
_NC, _NS = 2, 16
_NW = _NC * _NS
_RPW = _ROWS // _NW
_C = 32
_NCHUNK = _RPW // _C
_PC = _C // _BATCH
_LANES = 16
_SCALE = math.sqrt(_EMB)


def _sc_embed(tok_flat, table, pos_embedding):
    mesh = plsc.VectorSubcoreMesh(core_axis_name="c", subcore_axis_name="s")

    @functools.partial(
        pl.kernel,
        out_type=jax.ShapeDtypeStruct((_SEQ, _BATCH, _EMB), jnp.float32),
        mesh=mesh,
        scratch_types=[
            pltpu.VMEM((_RPW,), jnp.int32),
            pltpu.VMEM((2, _C, _EMB), jnp.float32),
            pltpu.VMEM((2, _PC, 1, _EMB), jnp.float32),
            pltpu.SemaphoreType.DMA,
            pltpu.SemaphoreType.DMA,
            pltpu.SemaphoreType.DMA,
        ],
    )
    def k(tok_hbm, table_hbm, pe_hbm, out_hbm, idx_v, rows2, pos2, gsem, psem, osem):
        wid = lax.axis_index("s") * _NC + lax.axis_index("c")
        base = wid * _RPW
        pltpu.sync_copy(tok_hbm.at[pl.ds(pl.multiple_of(base, _RPW), _RPW)], idx_v)

        def issue(g, slot):
            ioff = pl.multiple_of(g * _C, _C)
            off = pl.multiple_of(base + g * _C, _C)
            pltpu.async_copy(
                table_hbm.at[idx_v.at[pl.ds(ioff, _C)]], rows2.at[slot], gsem)
            poff = pl.multiple_of(off // _BATCH, _PC)
            pltpu.async_copy(pe_hbm.at[pl.ds(poff, _PC)], pos2.at[slot], psem)

        issue(0, 0)

        def wait_out():
            for s in range(_PC):
                pltpu.make_async_copy(
                    rows2.at[0, pl.ds(0, _BATCH)], out_hbm.at[0], osem).wait()

        def chunk(g, carry):
            b = lax.rem(g, 2)
            nxt = 1 - b

            @pl.when(g + 1 < _NCHUNK)
            def _prefetch():
                @pl.when(g >= 1)
                def _drain_prev_out():
                    wait_out()
                issue(g + 1, nxt)

            pltpu.make_async_copy(
                table_hbm.at[idx_v.at[pl.ds(0, _C)]], rows2.at[b], gsem).wait()
            pltpu.make_async_copy(
                pe_hbm.at[pl.ds(0, _PC)], pos2.at[b], psem).wait()

            def quad(q, c2):
                for j in range(_EMB // _LANES):
                    sl = pl.ds(j * _LANES, _LANES)
                    pv = pos2[b, q, 0, sl]
                    for t in range(_BATCH):
                        r = q * _BATCH + t
                        rows2[b, r, sl] = rows2[b, r, sl] * _SCALE + pv
                return c2

            lax.fori_loop(0, _PC, quad, 0)
            soff = (base + g * _C) // _BATCH
            for s in range(_PC):
                pltpu.async_copy(
                    rows2.at[b, pl.ds(s * _BATCH, _BATCH)],
                    out_hbm.at[soff + s], osem)
            return carry

        lax.fori_loop(0, _NCHUNK, chunk, 0)
        wait_out()
        wait_out()

    return k(tok_flat, table, pos_embedding)


def kernel(tokens, table, pos_embedding):
    tok_flat = tokens.reshape(-1).astype(jnp.int32)
    return _sc_embed(tok_flat, table, pos_embedding)


# speedup vs baseline: 2.0027x; 1.0179x over previous
import functools
import math

import jax
import jax.numpy as jnp
from jax import lax
from jax.experimental import pallas as pl
from jax.experimental.pallas import tpu as pltpu
from jax.experimental.pallas import tpu_sc as plsc

_EMB = 1024
_SEQ = 2048
_BATCH = 4
_ROWS = _SEQ * _BATCH
_NC, _NS = 2, 16
_NW = _NC * _NS
_RPW = _ROWS // _NW
_C = 32
_NCHUNK = _RPW // _C
_PC = _C // _BATCH
_LANES = 16
_UNROLL = 4
_SCALE = math.sqrt(_EMB)


def _sc_embed(tok_flat, table, pos_embedding):
    mesh = plsc.VectorSubcoreMesh(core_axis_name="c", subcore_axis_name="s")

    @functools.partial(
        pl.kernel,
        out_type=jax.ShapeDtypeStruct((_SEQ, _BATCH, _EMB), jnp.float32),
        mesh=mesh,
        scratch_types=[
            pltpu.VMEM((_RPW,), jnp.int32),
            pltpu.VMEM((2, _C, _EMB), jnp.float32),
            pltpu.VMEM((2, _PC, 1, _EMB), jnp.float32),
            pltpu.SemaphoreType.DMA,
            pltpu.SemaphoreType.DMA,
            pltpu.SemaphoreType.DMA,
        ],
    )
    def k(tok_hbm, table_hbm, pe_hbm, out_hbm, idx_v, rows2, pos2, gsem, psem, osem):
        wid = lax.axis_index("s") * _NC + lax.axis_index("c")
        base = wid * _RPW
        pltpu.sync_copy(tok_hbm.at[pl.ds(pl.multiple_of(base, _RPW), _RPW)], idx_v)

        def issue(g, slot):
            ioff = pl.multiple_of(g * _C, _C)
            off = pl.multiple_of(base + g * _C, _C)
            pltpu.async_copy(
                table_hbm.at[idx_v.at[pl.ds(ioff, _C)]], rows2.at[slot], gsem)
            poff = pl.multiple_of(off // _BATCH, _PC)
            pltpu.async_copy(pe_hbm.at[pl.ds(poff, _PC)], pos2.at[slot], psem)

        issue(0, 0)

        def wait_out():
            for s in range(_PC):
                pltpu.make_async_copy(
                    rows2.at[0, pl.ds(0, _BATCH)], out_hbm.at[0], osem).wait()

        def chunk(g, carry):
            b = lax.rem(g, 2)
            nxt = 1 - b

            @pl.when(g + 1 < _NCHUNK)
            def _prefetch():
                @pl.when(g >= 1)
                def _drain_prev_out():
                    wait_out()
                issue(g + 1, nxt)

            pltpu.make_async_copy(
                table_hbm.at[idx_v.at[pl.ds(0, _C)]], rows2.at[b], gsem).wait()
            pltpu.make_async_copy(
                pe_hbm.at[pl.ds(0, _PC)], pos2.at[b], psem).wait()

            def quad(q, c2):
                def jstep(jj, c3):
                    for ju in range(_UNROLL):
                        sl = pl.ds((jj * _UNROLL + ju) * _LANES, _LANES)
                        pv = pos2[b, q, 0, sl]
                        for t in range(_BATCH):
                            r = q * _BATCH + t
                            rows2[b, r, sl] = rows2[b, r, sl] * _SCALE + pv
                    return c3

                return lax.fori_loop(0, _EMB // _LANES // _UNROLL, jstep, c2)

            lax.fori_loop(0, _PC, quad, 0)
            soff = (base + g * _C) // _BATCH
            for s in range(_PC):
                pltpu.async_copy(
                    rows2.at[b, pl.ds(s * _BATCH, _BATCH)],
                    out_hbm.at[soff + s], osem)
            return carry

        lax.fori_loop(0, _NCHUNK, chunk, 0)
        wait_out()
        wait_out()

    return k(tok_flat, table, pos_embedding)


def kernel(tokens, table, pos_embedding):
    tok_flat = tokens.reshape(-1).astype(jnp.int32)
    return _sc_embed(tok_flat, table, pos_embedding)


# speedup vs baseline: 2.1186x; 1.0579x over previous
import functools
import math

import jax
import jax.numpy as jnp
from jax import lax
from jax.experimental import pallas as pl
from jax.experimental.pallas import tpu as pltpu
from jax.experimental.pallas import tpu_sc as plsc

_EMB = 1024
_SEQ = 2048
_BATCH = 4
_ROWS = _SEQ * _BATCH
_NC, _NS = 2, 16
_NW = _NC * _NS
_RPW = _ROWS // _NW
_C = 32
_NCHUNK = _RPW // _C
_PC = _C // _BATCH
_NBUF = 3
_LANES = 16
_UNROLL = 4
_SCALE = math.sqrt(_EMB)


def _sc_embed(tok_flat, table, pos_embedding):
    mesh = plsc.VectorSubcoreMesh(core_axis_name="c", subcore_axis_name="s")

    @functools.partial(
        pl.kernel,
        out_type=jax.ShapeDtypeStruct((_SEQ, _BATCH, _EMB), jnp.float32),
        mesh=mesh,
        scratch_types=[
            pltpu.VMEM((_RPW,), jnp.int32),
            pltpu.VMEM((_NBUF, _C, _EMB), jnp.float32),
            pltpu.VMEM((_NBUF, _PC, 1, _EMB), jnp.float32),
            pltpu.SemaphoreType.DMA,
            pltpu.SemaphoreType.DMA,
            pltpu.SemaphoreType.DMA,
        ],
    )
    def k(tok_hbm, table_hbm, pe_hbm, out_hbm, idx_v, rows3, pos3, gsem, psem, osem):
        wid = lax.axis_index("s") * _NC + lax.axis_index("c")
        base = wid * _RPW
        pltpu.sync_copy(tok_hbm.at[pl.ds(pl.multiple_of(base, _RPW), _RPW)], idx_v)

        def issue(g, slot):
            ioff = pl.multiple_of(g * _C, _C)
            off = pl.multiple_of(base + g * _C, _C)
            pltpu.async_copy(
                table_hbm.at[idx_v.at[pl.ds(ioff, _C)]], rows3.at[slot], gsem)
            poff = pl.multiple_of(off // _BATCH, _PC)
            pltpu.async_copy(pe_hbm.at[pl.ds(poff, _PC)], pos3.at[slot], psem)

        issue(0, 0)

        def wait_out():
            for s in range(_PC):
                pltpu.make_async_copy(
                    rows3.at[0, pl.ds(0, _BATCH)], out_hbm.at[0], osem).wait()

        def chunk(g, carry):
            b = lax.rem(g, _NBUF)

            @pl.when(g + 1 < _NCHUNK)
            def _prefetch():
                @pl.when(g >= _NBUF - 1)
                def _drain_old_out():
                    wait_out()
                issue(g + 1, lax.rem(g + 1, _NBUF))

            pltpu.make_async_copy(
                table_hbm.at[idx_v.at[pl.ds(0, _C)]], rows3.at[b], gsem).wait()
            pltpu.make_async_copy(
                pe_hbm.at[pl.ds(0, _PC)], pos3.at[b], psem).wait()

            def quad(q, c2):
                def jstep(jj, c3):
                    for ju in range(_UNROLL):
                        sl = pl.ds((jj * _UNROLL + ju) * _LANES, _LANES)
                        pv = pos3[b, q, 0, sl]
                        for t in range(_BATCH):
                            r = q * _BATCH + t
                            rows3[b, r, sl] = rows3[b, r, sl] * _SCALE + pv
                    return c3

                return lax.fori_loop(0, _EMB // _LANES // _UNROLL, jstep, c2)

            lax.fori_loop(0, _PC, quad, 0)
            soff = (base + g * _C) // _BATCH
            for s in range(_PC):
                pltpu.async_copy(
                    rows3.at[b, pl.ds(s * _BATCH, _BATCH)],
                    out_hbm.at[soff + s], osem)
            return carry

        lax.fori_loop(0, _NCHUNK, chunk, 0)
        for _ in range(_NBUF):
            wait_out()

    return k(tok_flat, table, pos_embedding)


def kernel(tokens, table, pos_embedding):
    tok_flat = tokens.reshape(-1).astype(jnp.int32)
    return _sc_embed(tok_flat, table, pos_embedding)


# speedup vs baseline: 2.2477x; 1.0610x over previous
import functools
import math

import jax
import jax.numpy as jnp
from jax import lax
from jax.experimental import pallas as pl
from jax.experimental.pallas import tpu as pltpu
from jax.experimental.pallas import tpu_sc as plsc

_EMB = 1024
_SEQ = 2048
_BATCH = 4
_ROWS = _SEQ * _BATCH
_NC, _NS = 2, 16
_NW = _NC * _NS
_RPW = _ROWS // _NW
_C = 16
_NCHUNK = _RPW // _C
_PC = _C // _BATCH
_NBUF = 6
_LANES = 16
_UNROLL = 4
_SCALE = math.sqrt(_EMB)


def _sc_embed(tok_flat, table, pos_embedding):
    mesh = plsc.VectorSubcoreMesh(core_axis_name="c", subcore_axis_name="s")

    @functools.partial(
        pl.kernel,
        out_type=jax.ShapeDtypeStruct((_SEQ, _BATCH, _EMB), jnp.float32),
        mesh=mesh,
        scratch_types=[
            pltpu.VMEM((_RPW,), jnp.int32),
            pltpu.VMEM((_NBUF, _C, _EMB), jnp.float32),
            pltpu.VMEM((_NBUF, _PC, 1, _EMB), jnp.float32),
            pltpu.SemaphoreType.DMA,
            pltpu.SemaphoreType.DMA,
            pltpu.SemaphoreType.DMA,
        ],
    )
    def k(tok_hbm, table_hbm, pe_hbm, out_hbm, idx_v, rows3, pos3, gsem, psem, osem):
        wid = lax.axis_index("s") * _NC + lax.axis_index("c")
        base = wid * _RPW
        pltpu.sync_copy(tok_hbm.at[pl.ds(pl.multiple_of(base, _RPW), _RPW)], idx_v)

        def issue(g, slot):
            ioff = pl.multiple_of(g * _C, _C)
            off = pl.multiple_of(base + g * _C, _C)
            pltpu.async_copy(
                table_hbm.at[idx_v.at[pl.ds(ioff, _C)]], rows3.at[slot], gsem)
            poff = pl.multiple_of(off // _BATCH, _PC)
            pltpu.async_copy(pe_hbm.at[pl.ds(poff, _PC)], pos3.at[slot], psem)

        issue(0, 0)

        def wait_out():
            for s in range(_PC):
                pltpu.make_async_copy(
                    rows3.at[0, pl.ds(0, _BATCH)], out_hbm.at[0], osem).wait()

        def chunk(g, carry):
            b = lax.rem(g, _NBUF)

            @pl.when(g + 1 < _NCHUNK)
            def _prefetch():
                @pl.when(g >= _NBUF - 1)
                def _drain_old_out():
                    wait_out()
                issue(g + 1, lax.rem(g + 1, _NBUF))

            pltpu.make_async_copy(
                table_hbm.at[idx_v.at[pl.ds(0, _C)]], rows3.at[b], gsem).wait()
            pltpu.make_async_copy(
                pe_hbm.at[pl.ds(0, _PC)], pos3.at[b], psem).wait()

            def quad(q, c2):
                def jstep(jj, c3):
                    for ju in range(_UNROLL):
                        sl = pl.ds((jj * _UNROLL + ju) * _LANES, _LANES)
                        pv = pos3[b, q, 0, sl]
                        for t in range(_BATCH):
                            r = q * _BATCH + t
                            rows3[b, r, sl] = rows3[b, r, sl] * _SCALE + pv
                    return c3

                return lax.fori_loop(0, _EMB // _LANES // _UNROLL, jstep, c2)

            lax.fori_loop(0, _PC, quad, 0)
            soff = (base + g * _C) // _BATCH
            for s in range(_PC):
                pltpu.async_copy(
                    rows3.at[b, pl.ds(s * _BATCH, _BATCH)],
                    out_hbm.at[soff + s], osem)
            return carry

        lax.fori_loop(0, _NCHUNK, chunk, 0)
        for _ in range(_NBUF):
            wait_out()

    return k(tok_flat, table, pos_embedding)


def kernel(tokens, table, pos_embedding):
    tok_flat = tokens.reshape(-1).astype(jnp.int32)
    return _sc_embed(tok_flat, table, pos_embedding)


# speedup vs baseline: 2.3319x; 1.0375x over previous
import functools
import math

import jax
import jax.numpy as jnp
from jax import lax
from jax.experimental import pallas as pl
from jax.experimental.pallas import tpu as pltpu
from jax.experimental.pallas import tpu_sc as plsc

_EMB = 1024
_SEQ = 2048
_BATCH = 4
_ROWS = _SEQ * _BATCH
_NC, _NS = 2, 16
_NW = _NC * _NS
_RPW = _ROWS // _NW
_C = 16
_NCHUNK = _RPW // _C
_PC = _C // _BATCH
_NBUF = 6
_LANES = 16
_UNROLL = 4
_SCALE = math.sqrt(_EMB)


def _sc_embed(tok_flat, table, pos_embedding):
    mesh = plsc.VectorSubcoreMesh(core_axis_name="c", subcore_axis_name="s")

    @functools.partial(
        pl.kernel,
        out_type=jax.ShapeDtypeStruct((_SEQ, _BATCH, _EMB), jnp.float32),
        mesh=mesh,
        scratch_types=[
            pltpu.VMEM((_RPW,), jnp.int32),
            pltpu.VMEM((_NBUF, _C, _EMB), jnp.float32),
            pltpu.VMEM((_NBUF, _PC, 1, _EMB), jnp.float32),
            pltpu.SemaphoreType.DMA,
            pltpu.SemaphoreType.DMA,
            pltpu.SemaphoreType.DMA,
        ],
    )
    def k(tok_hbm, table_hbm, pe_hbm, out_hbm, idx_v, rows3, pos3, gsem, psem, osem):
        wid = lax.axis_index("s") * _NC + lax.axis_index("c")
        base = wid * _RPW
        pltpu.sync_copy(tok_hbm.at[pl.ds(pl.multiple_of(base, _RPW), _RPW)], idx_v)

        def issue(g, slot):
            ioff = pl.multiple_of(g * _C, _C)
            off = pl.multiple_of(base + g * _C, _C)
            pltpu.async_copy(
                table_hbm.at[idx_v.at[pl.ds(ioff, _C)]], rows3.at[slot], gsem)
            poff = pl.multiple_of(off // _BATCH, _PC)
            pltpu.async_copy(pe_hbm.at[pl.ds(poff, _PC)], pos3.at[slot], psem)

        issue(0, 0)
        issue(1, 1)

        def wait_out():
            for s in range(_PC):
                pltpu.make_async_copy(
                    rows3.at[0, pl.ds(0, _BATCH)], out_hbm.at[0], osem).wait()

        def chunk(g, carry):
            b = lax.rem(g, _NBUF)

            @pl.when(g + 2 < _NCHUNK)
            def _prefetch():
                @pl.when(g >= _NBUF - 2)
                def _drain_old_out():
                    wait_out()
                issue(g + 2, lax.rem(g + 2, _NBUF))

            pltpu.make_async_copy(
                table_hbm.at[idx_v.at[pl.ds(0, _C)]], rows3.at[b], gsem).wait()
            pltpu.make_async_copy(
                pe_hbm.at[pl.ds(0, _PC)], pos3.at[b], psem).wait()

            def quad(q, c2):
                def jstep(jj, c3):
                    for ju in range(_UNROLL):
                        sl = pl.ds((jj * _UNROLL + ju) * _LANES, _LANES)
                        pv = pos3[b, q, 0, sl]
                        for t in range(_BATCH):
                            r = q * _BATCH + t
                            rows3[b, r, sl] = rows3[b, r, sl] * _SCALE + pv
                    return c3

                return lax.fori_loop(0, _EMB // _LANES // _UNROLL, jstep, c2)

            lax.fori_loop(0, _PC, quad, 0)
            soff = (base + g * _C) // _BATCH
            for s in range(_PC):
                pltpu.async_copy(
                    rows3.at[b, pl.ds(s * _BATCH, _BATCH)],
                    out_hbm.at[soff + s], osem)
            return carry

        lax.fori_loop(0, _NCHUNK, chunk, 0)
        for _ in range(_NBUF):
            wait_out()

    return k(tok_flat, table, pos_embedding)


def kernel(tokens, table, pos_embedding):
    tok_flat = tokens.reshape(-1).astype(jnp.int32)
    return _sc_embed(tok_flat, table, pos_embedding)


# speedup vs baseline: 2.3349x; 1.0013x over previous
import functools
import math

import jax
import jax.numpy as jnp
from jax import lax
from jax.experimental import pallas as pl
from jax.experimental.pallas import tpu as pltpu
from jax.experimental.pallas import tpu_sc as plsc

_EMB = 1024
_SEQ = 2048
_BATCH = 4
_ROWS = _SEQ * _BATCH
_NC, _NS = 2, 16
_NW = _NC * _NS
_RPW = _ROWS // _NW
_C = 16
_NCHUNK = _RPW // _C
_PC = _C // _BATCH
_NBUF = 6
_LANES = 16
_UNROLL = 4
_SCALE = math.sqrt(_EMB)


def _sc_embed(tok_flat, table, pos_embedding):
    mesh = plsc.VectorSubcoreMesh(core_axis_name="c", subcore_axis_name="s")

    @functools.partial(
        pl.kernel,
        out_type=jax.ShapeDtypeStruct((_SEQ, _BATCH, _EMB), jnp.float32),
        mesh=mesh,
        scratch_types=[
            pltpu.VMEM((_RPW,), jnp.int32),
            pltpu.VMEM((_NBUF, _C, _EMB), jnp.float32),
            pltpu.VMEM((_NBUF, _PC, 1, _EMB), jnp.float32),
            pltpu.SemaphoreType.DMA,
            pltpu.SemaphoreType.DMA,
            pltpu.SemaphoreType.DMA,
        ],
    )
    def k(tok_hbm, table_hbm, pe_hbm, out_hbm, idx_v, rows3, pos3, gsem, psem, osem):
        wid = lax.axis_index("s") * _NC + lax.axis_index("c")
        base = wid * _RPW
        pltpu.sync_copy(tok_hbm.at[pl.ds(pl.multiple_of(base, _RPW), _RPW)], idx_v)

        def issue(g, slot):
            ioff = pl.multiple_of(g * _C, _C)
            off = pl.multiple_of(base + g * _C, _C)
            pltpu.async_copy(
                table_hbm.at[idx_v.at[pl.ds(ioff, _C)]], rows3.at[slot], gsem)
            poff = pl.multiple_of(off // _BATCH, _PC)
            pltpu.async_copy(pe_hbm.at[pl.ds(poff, _PC)], pos3.at[slot], psem)

        issue(0, 0)
        issue(1, 1)
        issue(2, 2)

        def wait_out():
            for s in range(_PC):
                pltpu.make_async_copy(
                    rows3.at[0, pl.ds(0, _BATCH)], out_hbm.at[0], osem).wait()

        def chunk(g, carry):
            b = lax.rem(g, _NBUF)

            @pl.when(g + 3 < _NCHUNK)
            def _prefetch():
                @pl.when(g >= _NBUF - 3)
                def _drain_old_out():
                    wait_out()
                issue(g + 3, lax.rem(g + 3, _NBUF))

            pltpu.make_async_copy(
                table_hbm.at[idx_v.at[pl.ds(0, _C)]], rows3.at[b], gsem).wait()
            pltpu.make_async_copy(
                pe_hbm.at[pl.ds(0, _PC)], pos3.at[b], psem).wait()

            def quad(q, c2):
                def jstep(jj, c3):
                    for ju in range(_UNROLL):
                        sl = pl.ds((jj * _UNROLL + ju) * _LANES, _LANES)
                        pv = pos3[b, q, 0, sl]
                        for t in range(_BATCH):
                            r = q * _BATCH + t
                            rows3[b, r, sl] = rows3[b, r, sl] * _SCALE + pv
                    return c3

                return lax.fori_loop(0, _EMB // _LANES // _UNROLL, jstep, c2)

            lax.fori_loop(0, _PC, quad, 0)
            soff = (base + g * _C) // _BATCH
            for s in range(_PC):
                pltpu.async_copy(
                    rows3.at[b, pl.ds(s * _BATCH, _BATCH)],
                    out_hbm.at[soff + s], osem)
            return carry

        lax.fori_loop(0, _NCHUNK, chunk, 0)
        for _ in range(_NBUF):
            wait_out()

    return k(tok_flat, table, pos_embedding)


def kernel(tokens, table, pos_embedding):
    tok_flat = tokens.reshape(-1).astype(jnp.int32)
    return _sc_embed(tok_flat, table, pos_embedding)


# speedup vs baseline: 2.3420x; 1.0030x over previous
import functools
import math

import jax
import jax.numpy as jnp
from jax import lax
from jax.experimental import pallas as pl
from jax.experimental.pallas import tpu as pltpu
from jax.experimental.pallas import tpu_sc as plsc

_EMB = 1024
_SEQ = 2048
_BATCH = 4
_ROWS = _SEQ * _BATCH
_NC, _NS = 2, 16
_NW = _NC * _NS
_RPW = _ROWS // _NW
_C = 8
_NCHUNK = _RPW // _C
_PC = _C // _BATCH
_NBUF = 12
_LANES = 16
_UNROLL = 4
_SCALE = math.sqrt(_EMB)


def _sc_embed(tok_flat, table, pos_embedding):
    mesh = plsc.VectorSubcoreMesh(core_axis_name="c", subcore_axis_name="s")

    @functools.partial(
        pl.kernel,
        out_type=jax.ShapeDtypeStruct((_SEQ, _BATCH, _EMB), jnp.float32),
        mesh=mesh,
        scratch_types=[
            pltpu.VMEM((_RPW,), jnp.int32),
            pltpu.VMEM((_NBUF, _C, _EMB), jnp.float32),
            pltpu.VMEM((_NBUF, _PC, 1, _EMB), jnp.float32),
            pltpu.SemaphoreType.DMA,
            pltpu.SemaphoreType.DMA,
            pltpu.SemaphoreType.DMA,
        ],
    )
    def k(tok_hbm, table_hbm, pe_hbm, out_hbm, idx_v, rows3, pos3, gsem, psem, osem):
        wid = lax.axis_index("s") * _NC + lax.axis_index("c")
        base = wid * _RPW
        pltpu.sync_copy(tok_hbm.at[pl.ds(pl.multiple_of(base, _RPW), _RPW)], idx_v)

        def issue(g, slot):
            ioff = pl.multiple_of(g * _C, _C)
            off = pl.multiple_of(base + g * _C, _C)
            pltpu.async_copy(
                table_hbm.at[idx_v.at[pl.ds(ioff, _C)]], rows3.at[slot], gsem)
            poff = pl.multiple_of(off // _BATCH, _PC)
            pltpu.async_copy(pe_hbm.at[pl.ds(poff, _PC)], pos3.at[slot], psem)

        for p in range(4):
            issue(p, p)

        def wait_out():
            for s in range(_PC):
                pltpu.make_async_copy(
                    rows3.at[0, pl.ds(0, _BATCH)], out_hbm.at[0], osem).wait()

        def chunk(g, carry):
            b = lax.rem(g, _NBUF)

            @pl.when(g + 4 < _NCHUNK)
            def _prefetch():
                @pl.when(g >= _NBUF - 4)
                def _drain_old_out():
                    wait_out()
                issue(g + 4, lax.rem(g + 4, _NBUF))

            pltpu.make_async_copy(
                table_hbm.at[idx_v.at[pl.ds(0, _C)]], rows3.at[b], gsem).wait()
            pltpu.make_async_copy(
                pe_hbm.at[pl.ds(0, _PC)], pos3.at[b], psem).wait()

            def quad(q, c2):
                def jstep(jj, c3):
                    for ju in range(_UNROLL):
                        sl = pl.ds((jj * _UNROLL + ju) * _LANES, _LANES)
                        pv = pos3[b, q, 0, sl]
                        for t in range(_BATCH):
                            r = q * _BATCH + t
                            rows3[b, r, sl] = rows3[b, r, sl] * _SCALE + pv
                    return c3

                return lax.fori_loop(0, _EMB // _LANES // _UNROLL, jstep, c2)

            lax.fori_loop(0, _PC, quad, 0)
            soff = (base + g * _C) // _BATCH
            for s in range(_PC):
                pltpu.async_copy(
                    rows3.at[b, pl.ds(s * _BATCH, _BATCH)],
                    out_hbm.at[soff + s], osem)
            return carry

        lax.fori_loop(0, _NCHUNK, chunk, 0)
        for _ in range(_NBUF):
            wait_out()

    return k(tok_flat, table, pos_embedding)


def kernel(tokens, table, pos_embedding):
    tok_flat = tokens.reshape(-1).astype(jnp.int32)
    return _sc_embed(tok_flat, table, pos_embedding)


# speedup vs baseline: 2.3510x; 1.0038x over previous
import functools
import math

import jax
import jax.numpy as jnp
from jax import lax
from jax.experimental import pallas as pl
from jax.experimental.pallas import tpu as pltpu
from jax.experimental.pallas import tpu_sc as plsc

_EMB = 1024
_SEQ = 2048
_BATCH = 4
_ROWS = _SEQ * _BATCH
_NC, _NS = 2, 16
_NW = _NC * _NS
_RPW = _ROWS // _NW
_C = 8
_NCHUNK = _RPW // _C
_PC = _C // _BATCH
_NBUF = 12
_LANES = 16
_UNROLL = 4
_SCALE = math.sqrt(_EMB)


def _sc_embed(tok_flat, table, pos_embedding):
    mesh = plsc.VectorSubcoreMesh(core_axis_name="c", subcore_axis_name="s")

    @functools.partial(
        pl.kernel,
        out_type=jax.ShapeDtypeStruct((_SEQ, _BATCH, _EMB), jnp.float32),
        mesh=mesh,
        scratch_types=[
            pltpu.VMEM((_RPW,), jnp.int32),
            pltpu.VMEM((_NBUF, _C, _EMB), jnp.float32),
            pltpu.VMEM((_NBUF, _PC, 1, _EMB), jnp.float32),
            pltpu.SemaphoreType.DMA,
            pltpu.SemaphoreType.DMA,
            pltpu.SemaphoreType.DMA,
        ],
    )
    def k(tok_hbm, table_hbm, pe_hbm, out_hbm, idx_v, rows3, pos3, gsem, psem, osem):
        wid = lax.axis_index("s") * _NC + lax.axis_index("c")
        base = wid * _RPW
        pltpu.sync_copy(tok_hbm.at[pl.ds(pl.multiple_of(base, _RPW), _RPW)], idx_v)

        def issue(g, slot):
            ioff = pl.multiple_of(g * _C, _C)
            off = pl.multiple_of(base + g * _C, _C)
            pltpu.async_copy(
                table_hbm.at[idx_v.at[pl.ds(ioff, _C)]], rows3.at[slot], gsem)
            poff = pl.multiple_of(off // _BATCH, _PC)
            pltpu.async_copy(pe_hbm.at[pl.ds(poff, _PC)], pos3.at[slot], psem)

        for p in range(6):
            issue(p, p)

        def wait_out():
            for s in range(_PC):
                pltpu.make_async_copy(
                    rows3.at[0, pl.ds(0, _BATCH)], out_hbm.at[0], osem).wait()

        def chunk(g, carry):
            b = lax.rem(g, _NBUF)

            @pl.when(g + 6 < _NCHUNK)
            def _prefetch():
                @pl.when(g >= _NBUF - 6)
                def _drain_old_out():
                    wait_out()
                issue(g + 6, lax.rem(g + 6, _NBUF))

            pltpu.make_async_copy(
                table_hbm.at[idx_v.at[pl.ds(0, _C)]], rows3.at[b], gsem).wait()
            pltpu.make_async_copy(
                pe_hbm.at[pl.ds(0, _PC)], pos3.at[b], psem).wait()

            def quad(q, c2):
                def jstep(jj, c3):
                    for ju in range(_UNROLL):
                        sl = pl.ds((jj * _UNROLL + ju) * _LANES, _LANES)
                        pv = pos3[b, q, 0, sl]
                        for t in range(_BATCH):
                            r = q * _BATCH + t
                            rows3[b, r, sl] = rows3[b, r, sl] * _SCALE + pv
                    return c3

                return lax.fori_loop(0, _EMB // _LANES // _UNROLL, jstep, c2)

            lax.fori_loop(0, _PC, quad, 0)
            soff = (base + g * _C) // _BATCH
            for s in range(_PC):
                pltpu.async_copy(
                    rows3.at[b, pl.ds(s * _BATCH, _BATCH)],
                    out_hbm.at[soff + s], osem)
            return carry

        lax.fori_loop(0, _NCHUNK, chunk, 0)
        for _ in range(_NBUF):
            wait_out()

    return k(tok_flat, table, pos_embedding)


def kernel(tokens, table, pos_embedding):
    tok_flat = tokens.reshape(-1).astype(jnp.int32)
    return _sc_embed(tok_flat, table, pos_embedding)
